# Initial kernel scaffold; baseline (speedup 1.0000x reference)
import jax
import jax.numpy as jnp
from jax.experimental import pallas as pl


def kernel(input_val, W0a, b0a, W0b, b0b, W0c, b0c, W1pre, b1pre, W1int, b1int, W1out, b1out):
    raise NotImplementedError("write your pallas kernel here")


# speedup vs baseline: 1.3100x; 1.3100x over previous
import jax
import jax.numpy as jnp
from jax.experimental import pallas as pl
from jax.experimental.pallas import tpu as pltpu

N = 2048
E = 8


def _node0_body(x0_ref, oh0_ref, wa_ref, ba_ref, wb_ref, bb_ref, wc_ref, bc_ref, out_ref):
    e = pl.program_id(0)
    x0 = x0_ref[...]
    h = jnp.maximum(jnp.dot(x0, wa_ref[0], preferred_element_type=jnp.float32) + ba_ref[0], 0.0)
    h = jnp.maximum(jnp.dot(h, wb_ref[0], preferred_element_type=jnp.float32) + bb_ref[0], 0.0)
    h = jnp.maximum(jnp.dot(h, wc_ref[0], preferred_element_type=jnp.float32) + bc_ref[0], 0.0)
    col = jax.lax.broadcasted_iota(jnp.int32, (N, E), 1)
    w = jnp.sum(jnp.where(col == e, oh0_ref[...], 0.0), axis=1, keepdims=True)

    @pl.when(e == 0)
    def _():
        out_ref[...] = jnp.zeros_like(out_ref)

    out_ref[...] += w * h


def _node1_body(x1_ref, oh1_ref, prev_ref, wp_ref, bp_ref, wi_ref, bi_ref, wo_ref, bo_ref, out_ref):
    e = pl.program_id(0)
    x1 = x1_ref[...]
    p = jnp.maximum(jnp.dot(x1, wp_ref[0], preferred_element_type=jnp.float32) + bp_ref[0], 0.0)
    h1 = jnp.dot(prev_ref[...], wi_ref[0, 0:256, :], preferred_element_type=jnp.float32)
    h1 += jnp.dot(p, wi_ref[0, 256:768, :], preferred_element_type=jnp.float32)
    h1 = jnp.maximum(h1 + bi_ref[0], 0.0)
    o1 = jnp.dot(h1, wo_ref[0], preferred_element_type=jnp.float32) + bo_ref[0]
    col = jax.lax.broadcasted_iota(jnp.int32, (N, E), 1)
    w = jnp.sum(jnp.where(col == e, oh1_ref[...], 0.0), axis=1, keepdims=True)

    @pl.when(e == 0)
    def _():
        out_ref[...] = jnp.zeros_like(out_ref)

    out_ref[...] += w * o1


def kernel(input_val, W0a, b0a, W0b, b0b, W0c, b0c, W1pre, b1pre, W1int, b1int, W1out, b1out):
    x0 = input_val[:, 0:256]
    x1 = input_val[:, 256:512]
    oh0 = input_val[:, 512:520]
    oh1 = input_val[:, 520:528]

    b0a3 = b0a[:, None, :]
    b0b3 = b0b[:, None, :]
    b0c3 = b0c[:, None, :]
    b1pre3 = b1pre[:, None, :]
    b1int3 = b1int[:, None, :]
    b1out3 = b1out[:, None, :]

    full = lambda shape: pl.BlockSpec(shape, lambda e: tuple(0 for _ in shape))
    per_e_3d = lambda d1, d2: pl.BlockSpec((1, d1, d2), lambda e: (e, 0, 0))

    out0 = pl.pallas_call(
        _node0_body,
        grid=(E,),
        in_specs=[
            full((N, 256)),
            full((N, E)),
            per_e_3d(256, 512), per_e_3d(1, 512),
            per_e_3d(512, 512), per_e_3d(1, 512),
            per_e_3d(512, 256), per_e_3d(1, 256),
        ],
        out_specs=pl.BlockSpec((N, 256), lambda e: (0, 0)),
        out_shape=jax.ShapeDtypeStruct((N, 256), jnp.float32),
    )(x0, oh0, W0a, b0a3, W0b, b0b3, W0c, b0c3)

    out = pl.pallas_call(
        _node1_body,
        grid=(E,),
        in_specs=[
            full((N, 256)),
            full((N, E)),
            full((N, 256)),
            per_e_3d(256, 512), per_e_3d(1, 512),
            per_e_3d(768, 512), per_e_3d(1, 512),
            per_e_3d(512, 256), per_e_3d(1, 256),
        ],
        out_specs=pl.BlockSpec((N, 256), lambda e: (0, 0)),
        out_shape=jax.ShapeDtypeStruct((N, 256), jnp.float32),
    )(x1, oh1, out0, W1pre, b1pre3, W1int, b1int3, W1out, b1out3)

    return out
